# baseline (speedup 1.0000x reference)
import functools

import jax
import jax.numpy as jnp
from jax import lax
from jax.experimental import pallas as pl
from jax.experimental.pallas import tpu as pltpu
from jax.experimental.pallas import tpu_sc as plsc

L = 16
NW = 32
NSLOT = 8
SUB = 4
BLK = 128

_PERM_DNUMS = lax.GatherDimensionNumbers(
    offset_dims=(), collapsed_slice_dims=(0,), start_index_map=(0,))


def _permute(x, idx):
    return lax.gather(x, idx[:, None], _PERM_DNUMS, slice_sizes=(1,),
                      mode=lax.GatherScatterMode.PROMISE_IN_BOUNDS)


@functools.lru_cache(maxsize=None)
def _make_kernel(B: int, F: int):
    assert F == 2 * L
    b_per_w = B // NW
    n_groups = b_per_w // L
    mesh = plsc.VectorSubcoreMesh(core_axis_name="c", subcore_axis_name="s")

    @functools.partial(
        pl.kernel,
        mesh=mesh,
        out_type=jax.ShapeDtypeStruct((B,), jnp.float32),
        compiler_params=pltpu.CompilerParams(needs_layout_passes=False),
        scratch_types=[
            pltpu.VMEM((b_per_w,), jnp.int32),
            pltpu.VMEM((b_per_w,), jnp.int32),
            pltpu.VMEM((NSLOT, F, BLK), jnp.float32),
            pltpu.VMEM((NSLOT, F, BLK), jnp.float32),
            pltpu.VMEM((b_per_w,), jnp.float32),
            pltpu.SemaphoreType.DMA,
            pltpu.SemaphoreType.DMA,
            pltpu.SemaphoreType.DMA,
            pltpu.SemaphoreType.DMA,
        ],
    )
    def pmf_kernel(user_hbm, item_hbm, uids_hbm, iids_hbm, out_hbm,
                   uraw, iraw, gu, gi, outv,
                   su0, su1, sv0, sv1):
        sem_u = (su0, su1)
        sem_v = (sv0, sv1)
        wid = lax.axis_index("s") * 2 + lax.axis_index("c")
        base = wid * b_per_w
        pltpu.sync_copy(uids_hbm.at[pl.ds(base, b_per_w)], uraw)
        pltpu.sync_copy(iids_hbm.at[pl.ds(base, b_per_w)], iraw)

        lanes = lax.iota(jnp.int32, L)
        rows_lo = lanes
        rows_hi = lanes + L
        perms = [lanes ^ s for s in (1, 2, 4, 8)]
        masks = [(lanes & s) == 0 for s in (1, 2, 4, 8)]

        def group_body(g, carry):
            o = g * L
            idu = uraw[pl.ds(o, L)]
            idv = iraw[pl.ds(o, L)]
            cu = jnp.bitwise_and(idu, ~(BLK - 1))
            cv = jnp.bitwise_and(idv, ~(BLK - 1))
            lu = jnp.bitwise_and(idu, BLK - 1)
            lv = jnp.bitwise_and(idv, BLK - 1)

            def fire(k):
                p = k % 2
                for j in range(SUB):
                    i = k * SUB + j
                    s = p * SUB + j
                    pltpu.async_copy(
                        user_hbm.at[:, pl.ds(pl.multiple_of(cu[i], 128), BLK)],
                        gu.at[s], sem_u[p])
                    pltpu.async_copy(
                        item_hbm.at[:, pl.ds(pl.multiple_of(cv[i], 128), BLK)],
                        gi.at[s], sem_v[p])

            def drain_and_extract(k, vecs):
                p = k % 2
                for _ in range(SUB):
                    pltpu.make_async_copy(
                        user_hbm.at[:, pl.ds(0, BLK)],
                        gu.at[0], sem_u[p]).wait()
                    pltpu.make_async_copy(
                        item_hbm.at[:, pl.ds(0, BLK)],
                        gi.at[0], sem_v[p]).wait()
                for j in range(SUB):
                    i = k * SUB + j
                    s = p * SUB + j
                    lu_b = jnp.full((L,), lu[i], jnp.int32)
                    lv_b = jnp.full((L,), lv[i], jnp.int32)
                    u0 = plsc.load_gather(gu.at[s], [rows_lo, lu_b])
                    u1 = plsc.load_gather(gu.at[s], [rows_hi, lu_b])
                    v0 = plsc.load_gather(gi.at[s], [rows_lo, lv_b])
                    v1 = plsc.load_gather(gi.at[s], [rows_hi, lv_b])
                    vecs.append(u0 * v0 + u1 * v1)
                return vecs

            n_sub = L // SUB
            LAG = 1
            vecs = []
            for k in range(n_sub + LAG):
                if k < n_sub:
                    fire(k)
                if k >= LAG:
                    vecs = drain_and_extract(k - LAG, vecs)

            for lev in range(4):
                nxt = []
                for p in range(0, len(vecs), 2):
                    x, y = vecs[p], vecs[p + 1]
                    px = _permute(x, perms[lev])
                    py = _permute(y, perms[lev])
                    nxt.append(jnp.where(masks[lev], x + px, y + py))
                vecs = nxt
            outv[pl.ds(o, L)] = vecs[0]
            return carry

        lax.fori_loop(0, n_groups, group_body, 0)
        pltpu.sync_copy(outv, out_hbm.at[pl.ds(base, b_per_w)])

    return pmf_kernel


def kernel(user_table, item_table, user_ids, item_ids):
    B = user_ids.shape[0]
    F = user_table.shape[1]
    k = _make_kernel(B, F)
    return k(user_table.T, item_table.T,
             user_ids.astype(jnp.int32), item_ids.astype(jnp.int32))


# speedup vs baseline: 1.0143x; 1.0143x over previous
import functools

import jax
import jax.numpy as jnp
from jax import lax
from jax.experimental import pallas as pl
from jax.experimental.pallas import tpu as pltpu
from jax.experimental.pallas import tpu_sc as plsc

L = 16
NW = 32
NSLOT = 12
SUB = 4
BLK = 128

_PERM_DNUMS = lax.GatherDimensionNumbers(
    offset_dims=(), collapsed_slice_dims=(0,), start_index_map=(0,))


def _permute(x, idx):
    return lax.gather(x, idx[:, None], _PERM_DNUMS, slice_sizes=(1,),
                      mode=lax.GatherScatterMode.PROMISE_IN_BOUNDS)


@functools.lru_cache(maxsize=None)
def _make_kernel(B: int, F: int):
    assert F == 2 * L
    b_per_w = B // NW
    n_groups = b_per_w // L
    mesh = plsc.VectorSubcoreMesh(core_axis_name="c", subcore_axis_name="s")

    @functools.partial(
        pl.kernel,
        mesh=mesh,
        out_type=jax.ShapeDtypeStruct((B,), jnp.float32),
        compiler_params=pltpu.CompilerParams(needs_layout_passes=False),
        scratch_types=[
            pltpu.VMEM((b_per_w,), jnp.int32),
            pltpu.VMEM((b_per_w,), jnp.int32),
            pltpu.VMEM((NSLOT, F, BLK), jnp.float32),
            pltpu.VMEM((NSLOT, F, BLK), jnp.float32),
            pltpu.VMEM((b_per_w,), jnp.float32),
            pltpu.SemaphoreType.DMA,
            pltpu.SemaphoreType.DMA,
            pltpu.SemaphoreType.DMA,
            pltpu.SemaphoreType.DMA,
            pltpu.SemaphoreType.DMA,
            pltpu.SemaphoreType.DMA,
        ],
    )
    def pmf_kernel(user_hbm, item_hbm, uids_hbm, iids_hbm, out_hbm,
                   uraw, iraw, gu, gi, outv,
                   su0, su1, su2, sv0, sv1, sv2):
        sem_u = (su0, su1, su2)
        sem_v = (sv0, sv1, sv2)
        wid = lax.axis_index("s") * 2 + lax.axis_index("c")
        base = wid * b_per_w
        pltpu.sync_copy(uids_hbm.at[pl.ds(base, b_per_w)], uraw)
        pltpu.sync_copy(iids_hbm.at[pl.ds(base, b_per_w)], iraw)

        lanes = lax.iota(jnp.int32, L)
        rows_lo = lanes
        rows_hi = lanes + L
        perms = [lanes ^ s for s in (1, 2, 4, 8)]
        masks = [(lanes & s) == 0 for s in (1, 2, 4, 8)]

        def group_body(g, carry):
            o = g * L
            idu = uraw[pl.ds(o, L)]
            idv = iraw[pl.ds(o, L)]
            cu = jnp.bitwise_and(idu, ~(BLK - 1))
            cv = jnp.bitwise_and(idv, ~(BLK - 1))
            lu = jnp.bitwise_and(idu, BLK - 1)
            lv = jnp.bitwise_and(idv, BLK - 1)

            def fire(k):
                p = k % 3
                for j in range(SUB):
                    i = k * SUB + j
                    s = p * SUB + j
                    pltpu.async_copy(
                        user_hbm.at[:, pl.ds(pl.multiple_of(cu[i], 128), BLK)],
                        gu.at[s], sem_u[p])
                    pltpu.async_copy(
                        item_hbm.at[:, pl.ds(pl.multiple_of(cv[i], 128), BLK)],
                        gi.at[s], sem_v[p])

            def drain_and_extract(k, vecs):
                p = k % 3
                for _ in range(SUB):
                    pltpu.make_async_copy(
                        user_hbm.at[:, pl.ds(0, BLK)],
                        gu.at[0], sem_u[p]).wait()
                    pltpu.make_async_copy(
                        item_hbm.at[:, pl.ds(0, BLK)],
                        gi.at[0], sem_v[p]).wait()
                for j in range(SUB):
                    i = k * SUB + j
                    s = p * SUB + j
                    lu_b = jnp.full((L,), lu[i], jnp.int32)
                    lv_b = jnp.full((L,), lv[i], jnp.int32)
                    u0 = plsc.load_gather(gu.at[s], [rows_lo, lu_b])
                    u1 = plsc.load_gather(gu.at[s], [rows_hi, lu_b])
                    v0 = plsc.load_gather(gi.at[s], [rows_lo, lv_b])
                    v1 = plsc.load_gather(gi.at[s], [rows_hi, lv_b])
                    vecs.append(u0 * v0 + u1 * v1)
                return vecs

            n_sub = L // SUB
            LAG = 2
            vecs = []
            for k in range(n_sub + LAG):
                if k < n_sub:
                    fire(k)
                if k >= LAG:
                    vecs = drain_and_extract(k - LAG, vecs)

            for lev in range(4):
                nxt = []
                for p in range(0, len(vecs), 2):
                    x, y = vecs[p], vecs[p + 1]
                    px = _permute(x, perms[lev])
                    py = _permute(y, perms[lev])
                    nxt.append(jnp.where(masks[lev], x + px, y + py))
                vecs = nxt
            outv[pl.ds(o, L)] = vecs[0]
            return carry

        lax.fori_loop(0, n_groups, group_body, 0)
        pltpu.sync_copy(outv, out_hbm.at[pl.ds(base, b_per_w)])

    return pmf_kernel


def kernel(user_table, item_table, user_ids, item_ids):
    B = user_ids.shape[0]
    F = user_table.shape[1]
    k = _make_kernel(B, F)
    return k(user_table.T, item_table.T,
             user_ids.astype(jnp.int32), item_ids.astype(jnp.int32))


# speedup vs baseline: 1.2292x; 1.2119x over previous
import functools

import jax
import jax.numpy as jnp
from jax import lax
from jax.experimental import pallas as pl
from jax.experimental.pallas import tpu as pltpu
from jax.experimental.pallas import tpu_sc as plsc

L = 16
NW = 32
SUB = 2
NPAR = 4
NSLOT = NPAR * SUB
BLK = 128
PRE = 3

_PERM_DNUMS = lax.GatherDimensionNumbers(
    offset_dims=(), collapsed_slice_dims=(0,), start_index_map=(0,))


def _permute(x, idx):
    return lax.gather(x, idx[:, None], _PERM_DNUMS, slice_sizes=(1,),
                      mode=lax.GatherScatterMode.PROMISE_IN_BOUNDS)


@functools.lru_cache(maxsize=None)
def _make_kernel(B: int, F: int):
    assert F == 2 * L
    b_per_w = B // NW
    n_groups = b_per_w // L
    n_sub = L // SUB
    assert PRE < NPAR and PRE < n_sub
    mesh = plsc.VectorSubcoreMesh(core_axis_name="c", subcore_axis_name="s")

    @functools.partial(
        pl.kernel,
        mesh=mesh,
        out_type=jax.ShapeDtypeStruct((B,), jnp.float32),
        compiler_params=pltpu.CompilerParams(needs_layout_passes=False),
        scratch_types=[
            pltpu.VMEM((b_per_w,), jnp.int32),
            pltpu.VMEM((b_per_w,), jnp.int32),
            pltpu.VMEM((NSLOT, F, BLK), jnp.float32),
            pltpu.VMEM((NSLOT, F, BLK), jnp.float32),
            pltpu.VMEM((b_per_w,), jnp.float32),
            pltpu.SemaphoreType.DMA,
            pltpu.SemaphoreType.DMA,
            pltpu.SemaphoreType.DMA,
            pltpu.SemaphoreType.DMA,
            pltpu.SemaphoreType.DMA,
            pltpu.SemaphoreType.DMA,
            pltpu.SemaphoreType.DMA,
            pltpu.SemaphoreType.DMA,
        ],
    )
    def pmf_kernel(user_hbm, item_hbm, uids_hbm, iids_hbm, out_hbm,
                   uraw, iraw, gu, gi, outv,
                   su0, su1, su2, su3, sv0, sv1, sv2, sv3):
        sem_u = (su0, su1, su2, su3)
        sem_v = (sv0, sv1, sv2, sv3)
        wid = lax.axis_index("s") * 2 + lax.axis_index("c")
        base = wid * b_per_w
        pltpu.sync_copy(uids_hbm.at[pl.ds(base, b_per_w)], uraw)
        pltpu.sync_copy(iids_hbm.at[pl.ds(base, b_per_w)], iraw)

        lanes = lax.iota(jnp.int32, L)
        rows_lo = lanes
        rows_hi = lanes + L
        perms = [lanes ^ s for s in (1, 2, 4, 8)]
        masks = [(lanes & s) == 0 for s in (1, 2, 4, 8)]

        def load_cols(g):
            o = g * L
            cu = jnp.bitwise_and(uraw[pl.ds(o, L)], ~(BLK - 1))
            cv = jnp.bitwise_and(iraw[pl.ds(o, L)], ~(BLK - 1))
            return cu, cv

        def fire(cu, cv, k):
            p = k % NPAR
            for j in range(SUB):
                i = k * SUB + j
                s = p * SUB + j
                pltpu.async_copy(
                    user_hbm.at[:, pl.ds(pl.multiple_of(cu[i], BLK), BLK)],
                    gu.at[s], sem_u[p])
                pltpu.async_copy(
                    item_hbm.at[:, pl.ds(pl.multiple_of(cv[i], BLK), BLK)],
                    gi.at[s], sem_v[p])

        def drain_and_extract(lu, lv, k, vecs):
            p = k % NPAR
            for _ in range(SUB):
                pltpu.make_async_copy(
                    user_hbm.at[:, pl.ds(0, BLK)],
                    gu.at[0], sem_u[p]).wait()
                pltpu.make_async_copy(
                    item_hbm.at[:, pl.ds(0, BLK)],
                    gi.at[0], sem_v[p]).wait()
            for j in range(SUB):
                i = k * SUB + j
                s = p * SUB + j
                lu_b = jnp.full((L,), lu[i], jnp.int32)
                lv_b = jnp.full((L,), lv[i], jnp.int32)
                u0 = plsc.load_gather(gu.at[s], [rows_lo, lu_b])
                u1 = plsc.load_gather(gu.at[s], [rows_hi, lu_b])
                v0 = plsc.load_gather(gi.at[s], [rows_lo, lv_b])
                v1 = plsc.load_gather(gi.at[s], [rows_hi, lv_b])
                vecs.append(u0 * v0 + u1 * v1)
            return vecs

        def group_body(g, carry):
            o = g * L
            lu = jnp.bitwise_and(uraw[pl.ds(o, L)], BLK - 1)
            lv = jnp.bitwise_and(iraw[pl.ds(o, L)], BLK - 1)
            cu, cv = load_cols(g)

            vecs = []
            for k in range(PRE, n_sub):
                fire(cu, cv, k)
                vecs = drain_and_extract(lu, lv, k - PRE, vecs)

            for idx in range(PRE):
                k = n_sub - PRE + idx

                def fire_one_next(idx=idx):
                    cun, cvn = load_cols(g + 1)
                    fire(cun, cvn, idx)

                pl.when(g + 1 < n_groups)(fire_one_next)
                vecs = drain_and_extract(lu, lv, k, vecs)

            for lev in range(4):
                nxt = []
                for p in range(0, len(vecs), 2):
                    x, y = vecs[p], vecs[p + 1]
                    px = _permute(x, perms[lev])
                    py = _permute(y, perms[lev])
                    nxt.append(jnp.where(masks[lev], x + px, y + py))
                vecs = nxt
            outv[pl.ds(o, L)] = vecs[0]
            return carry

        cu0, cv0 = load_cols(0)
        for k in range(PRE):
            fire(cu0, cv0, k)
        lax.fori_loop(0, n_groups, group_body, 0)
        pltpu.sync_copy(outv, out_hbm.at[pl.ds(base, b_per_w)])

    return pmf_kernel


def kernel(user_table, item_table, user_ids, item_ids):
    B = user_ids.shape[0]
    F = user_table.shape[1]
    k = _make_kernel(B, F)
    return k(user_table.T, item_table.T,
             user_ids.astype(jnp.int32), item_ids.astype(jnp.int32))
